# baseline (speedup 1.0000x reference)
import jax
import jax.numpy as jnp
from jax import lax
from jax.experimental import pallas as pl
from jax.experimental.pallas import tpu as pltpu
from jax.experimental.pallas import tpu_sc as plsc

N = 10000
E = 320000
D = 128
D_OUT = 16

NC = 2
NS = 16
NW = NC * NS
CHUNK = 64
NCHUNKS = 5120
EPAD = NCHUNKS * CHUNK
CH_PER_W = NCHUNKS // NW
IDXB = 8
NACC = 10112
ROWS_PER_SUB = NACC // NS

CCHUNK = 128
CNCHUNKS = EPAD // CCHUNK
CCH_PER_W = CNCHUNKS // NW


def _fill_zeros(ref, rows, width):
    zf = jnp.zeros((16,), jnp.float32)

    @pl.loop(0, rows)
    def _(r):
        @pl.loop(0, width, step=16)
        def _(c0):
            ref[r, pl.ds(c0, 16)] = zf


def _zero_acc(acc, zeros_v, sid):
    blk = zeros_v.shape[0]

    @pl.loop(0, ROWS_PER_SUB // blk)
    def _(t):
        pltpu.sync_copy(zeros_v, acc.at[pl.ds(sid * ROWS_PER_SUB + t * blk, blk)])
    rem = ROWS_PER_SUB % blk
    if rem:
        off = sid * ROWS_PER_SUB + (ROWS_PER_SUB // blk) * blk
        pltpu.sync_copy(zeros_v.at[pl.ds(0, rem)], acc.at[pl.ds(off, rem)])


def _sc_gather_sum_body(h_hbm, src_hbm, dst_hbm, s_out, acc,
                        src_v, dst_v, b0, b1, b2, b3,
                        g0, g1, g2, g3, a0, a1, a2, a3):
    cid = lax.axis_index("c")
    sid = lax.axis_index("s")
    wid = sid * NC + cid

    bufs = (b0, b1, b2, b3)
    gsem = (g0, g1, g2, g3)
    asem = (a0, a1, a2, a3)

    _fill_zeros(b0, CHUNK, D)
    _zero_acc(acc, b0, sid)
    plsc.subcore_barrier()

    @pl.loop(0, CH_PER_W // IDXB)
    def _(t):
        row0 = wid * CH_PER_W + t * IDXB
        pltpu.sync_copy(src_hbm.at[pl.ds(row0, IDXB)], src_v)
        pltpu.sync_copy(dst_hbm.at[pl.ds(row0, IDXB)], dst_v)
        pend_g = [None] * 4
        pend_a = [None] * 4
        for j in range(3):
            pend_g[j] = pltpu.async_copy(h_hbm.at[src_v.at[j]], bufs[j],
                                         gsem[j])
        for j in range(IDXB):
            if j + 3 < IDXB:
                s = (j + 3) % 4
                if pend_a[s] is not None:
                    pend_a[s].wait()
                pend_g[s] = pltpu.async_copy(h_hbm.at[src_v.at[j + 3]],
                                             bufs[s], gsem[s])
            b = j % 4
            pend_g[b].wait()
            pend_a[b] = pltpu.async_copy(bufs[b], acc.at[dst_v.at[j]],
                                         asem[b], add=True)
        for b in range(4):
            pend_a[b].wait()

    plsc.subcore_barrier()

    off = sid * ROWS_PER_SUB
    pltpu.sync_copy(acc.at[pl.ds(off, ROWS_PER_SUB)],
                    s_out.at[cid, pl.ds(off, ROWS_PER_SUB)])


def _make_sc_gather_sum():
    mesh = plsc.VectorSubcoreMesh(core_axis_name="c", subcore_axis_name="s")
    return pl.kernel(
        _sc_gather_sum_body,
        out_type=jax.ShapeDtypeStruct((NC, NACC, D), jnp.float32),
        mesh=mesh,
        scratch_types=[
            pltpu.VMEM_SHARED((NACC, D), jnp.float32),
            pltpu.VMEM((IDXB, CHUNK), jnp.int32),
            pltpu.VMEM((IDXB, CHUNK), jnp.int32),
            pltpu.VMEM((CHUNK, D), jnp.float32),
            pltpu.VMEM((CHUNK, D), jnp.float32),
            pltpu.VMEM((CHUNK, D), jnp.float32),
            pltpu.VMEM((CHUNK, D), jnp.float32),
            pltpu.SemaphoreType.DMA,
            pltpu.SemaphoreType.DMA,
            pltpu.SemaphoreType.DMA,
            pltpu.SemaphoreType.DMA,
            pltpu.SemaphoreType.DMA,
            pltpu.SemaphoreType.DMA,
            pltpu.SemaphoreType.DMA,
            pltpu.SemaphoreType.DMA,
        ],
    )


def _sc_count_body(dst_hbm, c_out, acc, dst_v, ones_v, zeros_v):
    cid = lax.axis_index("c")
    sid = lax.axis_index("s")
    wid = sid * NC + cid

    _fill_zeros(zeros_v, CCHUNK, D)
    of = jnp.ones((16,), jnp.float32)

    @pl.loop(0, CCHUNK)
    def _(r):
        @pl.loop(0, D, step=16)
        def _(c0):
            ones_v[r, pl.ds(c0, 16)] = of

    _zero_acc(acc, zeros_v, sid)
    plsc.subcore_barrier()

    @pl.loop(0, CCH_PER_W // IDXB)
    def _(t):
        row0 = wid * CCH_PER_W + t * IDXB
        pltpu.sync_copy(dst_hbm.at[pl.ds(row0, IDXB)], dst_v)
        for j in range(IDXB):
            pltpu.sync_copy(ones_v, acc.at[dst_v.at[j]], add=True)

    plsc.subcore_barrier()

    off = sid * ROWS_PER_SUB
    pltpu.sync_copy(acc.at[pl.ds(off, ROWS_PER_SUB)],
                    c_out.at[cid, pl.ds(off, ROWS_PER_SUB)])


def _make_sc_count():
    mesh = plsc.VectorSubcoreMesh(core_axis_name="c", subcore_axis_name="s")
    return pl.kernel(
        _sc_count_body,
        out_type=jax.ShapeDtypeStruct((NC, NACC, D), jnp.float32),
        mesh=mesh,
        scratch_types=[
            pltpu.VMEM_SHARED((NACC, D), jnp.float32),
            pltpu.VMEM((IDXB, CCHUNK), jnp.int32),
            pltpu.VMEM((CCHUNK, D), jnp.float32),
            pltpu.VMEM((CCHUNK, D), jnp.float32),
        ],
    )


def _layer_math(s_ref, c_ref, h_ref, wl_ref, wr_ref, b_ref):
    s = s_ref[0] + s_ref[1]
    cnt = c_ref[0, :, 0:1] + c_ref[1, :, 0:1]
    agg = s / jnp.maximum(cnt, 1.0)
    t = (jnp.dot(agg, wl_ref[...], preferred_element_type=jnp.float32,
                 precision=lax.Precision.HIGHEST)
         + jnp.dot(h_ref[...], wr_ref[...], preferred_element_type=jnp.float32,
                   precision=lax.Precision.HIGHEST)
         + b_ref[...])
    return jnp.where(t >= 0, t, 0.01 * t)


def _combine_body(s_ref, c_ref, h_ref, wl_ref, wr_ref, b_ref, o_ref):
    o_ref[...] = _layer_math(s_ref, c_ref, h_ref, wl_ref, wr_ref, b_ref)


BLK = 1000
NBLK = N // BLK


def _combine(s_part, c_part, h, wl, wr, b):
    return pl.pallas_call(
        _combine_body,
        grid=(NBLK,),
        in_specs=[
            pl.BlockSpec((NC, BLK, D), lambda i: (0, i, 0)),
            pl.BlockSpec((NC, BLK, 16), lambda i: (0, i, 0)),
            pl.BlockSpec((BLK, D), lambda i: (i, 0)),
            pl.BlockSpec((D, D), lambda i: (0, 0)),
            pl.BlockSpec((D, D), lambda i: (0, 0)),
            pl.BlockSpec((1, D), lambda i: (0, 0)),
        ],
        out_specs=pl.BlockSpec((BLK, D), lambda i: (i, 0)),
        out_shape=jax.ShapeDtypeStruct((N, D), jnp.float32),
    )(s_part, c_part, h, wl, wr, b)


def _combine_final_body(s_ref, c_ref, h_ref, wl_ref, wr_ref, b_ref,
                        wfc_ref, bfc_ref, o_ref, mx_ref):
    i = pl.program_id(0)

    @pl.when(i == 0)
    def _():
        mx_ref[...] = jnp.full((8, D), -jnp.inf, jnp.float32)

    hb = _layer_math(s_ref, c_ref, h_ref, wl_ref, wr_ref, b_ref)
    mx_ref[...] = jnp.maximum(mx_ref[...],
                              jnp.max(hb.reshape(BLK // 8, 8, D), axis=0))

    @pl.when(i == NBLK - 1)
    def _():
        pooled = jnp.max(mx_ref[...], axis=0).reshape(1, D)
        o_ref[...] = (jnp.dot(pooled, wfc_ref[...],
                              preferred_element_type=jnp.float32,
                              precision=lax.Precision.HIGHEST)
                      + bfc_ref[...])


def _combine_final(s_part, c_part, h, wl, wr, b, wfc, bfc):
    return pl.pallas_call(
        _combine_final_body,
        grid=(NBLK,),
        in_specs=[
            pl.BlockSpec((NC, BLK, D), lambda i: (0, i, 0)),
            pl.BlockSpec((NC, BLK, 16), lambda i: (0, i, 0)),
            pl.BlockSpec((BLK, D), lambda i: (i, 0)),
            pl.BlockSpec((D, D), lambda i: (0, 0)),
            pl.BlockSpec((D, D), lambda i: (0, 0)),
            pl.BlockSpec((1, D), lambda i: (0, 0)),
            pl.BlockSpec((D, D_OUT), lambda i: (0, 0)),
            pl.BlockSpec((1, D_OUT), lambda i: (0, 0)),
        ],
        out_specs=pl.BlockSpec((1, D_OUT), lambda i: (0, 0)),
        out_shape=jax.ShapeDtypeStruct((1, D_OUT), jnp.float32),
        scratch_shapes=[pltpu.VMEM((8, D), jnp.float32)],
    )(s_part, c_part, h, wl, wr, b, wfc, bfc)


def kernel(x, edge_index, batch, W1l, W1r, b1, W2l, W2r, b2, W3l, W3r, b3,
           W_fc, b_fc):
    src = edge_index[0]
    dst = edge_index[1]
    pad = EPAD - E
    src2d = jnp.concatenate(
        [src, jnp.zeros((pad,), jnp.int32)]).reshape(NCHUNKS, CHUNK)
    dst2d = jnp.concatenate(
        [dst, jnp.full((pad,), N, jnp.int32)]).reshape(NCHUNKS, CHUNK)
    dst2d_w = dst2d.reshape(CNCHUNKS, CCHUNK)

    gather_sum = _make_sc_gather_sum()
    count = _make_sc_count()

    c_full = count(dst2d_w)
    c_part = c_full[:, :, :16]

    s1 = gather_sum(x, src2d, dst2d)
    h1 = _combine(s1, c_part, x, W1l, W1r, b1.reshape(1, D))
    s2 = gather_sum(h1, src2d, dst2d)
    h2 = _combine(s2, c_part, h1, W2l, W2r, b2.reshape(1, D))
    s3 = gather_sum(h2, src2d, dst2d)
    return _combine_final(s3, c_part, h2, W3l, W3r, b3.reshape(1, D),
                          W_fc, b_fc.reshape(1, D_OUT))


# speedup vs baseline: 1.4119x; 1.4119x over previous
import jax
import jax.numpy as jnp
from jax import lax
from jax.experimental import pallas as pl
from jax.experimental.pallas import tpu as pltpu
from jax.experimental.pallas import tpu_sc as plsc

N = 10000
E = 320000
D = 128
DH = 64
D_OUT = 16

NC = 2
NS = 16
NW = NC * NS
CHUNK = 64
NCHUNKS = 5120
EPAD = NCHUNKS * CHUNK
CH_PER_T = NCHUNKS // NS
IDXB = 8
NACC = 10112
ROWS_PER_SUB = NACC // NS

CCHUNK = 128
CNCHUNKS = EPAD // CCHUNK
CCH_PER_W = CNCHUNKS // NW


def _fill_zeros(ref, rows, width):
    zf = jnp.zeros((16,), jnp.float32)

    @pl.loop(0, rows)
    def _(r):
        @pl.loop(0, width, step=16)
        def _(c0):
            ref[r, pl.ds(c0, 16)] = zf


def _zero_acc(acc, zeros_v, sid):
    blk = zeros_v.shape[0]

    @pl.loop(0, ROWS_PER_SUB // blk)
    def _(t):
        pltpu.sync_copy(zeros_v, acc.at[pl.ds(sid * ROWS_PER_SUB + t * blk, blk)])
    rem = ROWS_PER_SUB % blk
    if rem:
        off = sid * ROWS_PER_SUB + (ROWS_PER_SUB // blk) * blk
        pltpu.sync_copy(zeros_v.at[pl.ds(0, rem)], acc.at[pl.ds(off, rem)])


def _copy_half(srcb, dstb, rows, src_lane0, dst_lane0):
    @pl.loop(0, rows)
    def _(r):
        for k in range(DH // 16):
            dstb[r, pl.ds(dst_lane0 + k * 16, 16)] = (
                srcb[r, pl.ds(src_lane0 + k * 16, 16)])


def _sc_gather_sum_body(h_hbm, src_hbm, dst_hbm, s_out, work,
                        src_v, dst_v, bg0, bg1, ba0, ba1, sb,
                        g0, g1, a0, a1):
    cid = lax.axis_index("c")
    sid = lax.axis_index("s")

    bg = (bg0, bg1)
    ba = (ba0, ba1)
    gsem = (g0, g1)
    asem = (a0, a1)

    off = sid * ROWS_PER_SUB
    nfull = ROWS_PER_SUB // CHUNK
    rem = ROWS_PER_SUB % CHUNK

    _fill_zeros(ba0, CHUNK, D)

    @pl.loop(0, nfull)
    def _(bk):
        r0 = off + bk * CHUNK
        pltpu.sync_copy(h_hbm.at[pl.ds(r0, CHUNK)], bg0)

        @pl.when(cid == 0)
        def _():
            _copy_half(bg0, ba0, CHUNK, 0, 0)

        @pl.when(cid == 1)
        def _():
            _copy_half(bg0, ba0, CHUNK, DH, 0)
        pltpu.sync_copy(ba0, work.at[pl.ds(r0, CHUNK)])
    if rem:
        r0 = off + nfull * CHUNK
        pltpu.sync_copy(h_hbm.at[pl.ds(r0, rem)], bg0.at[pl.ds(0, rem)])

        @pl.when(cid == 0)
        def _():
            _copy_half(bg0, ba0, rem, 0, 0)

        @pl.when(cid == 1)
        def _():
            _copy_half(bg0, ba0, rem, DH, 0)
        pltpu.sync_copy(ba0.at[pl.ds(0, rem)], work.at[pl.ds(r0, rem)])

    _fill_zeros(ba0, CHUNK, D)
    _fill_zeros(ba1, CHUNK, D)
    plsc.subcore_barrier()

    @pl.loop(0, CH_PER_T // IDXB)
    def _(t):
        row0 = sid * CH_PER_T + t * IDXB
        pltpu.sync_copy(src_hbm.at[pl.ds(row0, IDXB)], src_v)
        pltpu.sync_copy(dst_hbm.at[pl.ds(row0, IDXB)], dst_v)
        pend_g = [None, None]
        pend_a = [None, None]
        pend_g[0] = pltpu.async_copy(work.at[src_v.at[0]], bg0, g0)
        for j in range(IDXB):
            b = j % 2
            if j + 1 < IDXB:
                nb = (j + 1) % 2
                pend_g[nb] = pltpu.async_copy(work.at[src_v.at[j + 1]],
                                              bg[nb], gsem[nb])
            pend_g[b].wait()
            if pend_a[b] is not None:
                pend_a[b].wait()
            _copy_half(bg[b], ba[b], CHUNK, 0, DH)
            pend_a[b] = pltpu.async_copy(ba[b], work.at[dst_v.at[j]],
                                         asem[b], add=True)
        pend_a[0].wait()
        pend_a[1].wait()

    plsc.subcore_barrier()

    @pl.loop(0, nfull)
    def _(bk):
        r0 = off + bk * CHUNK
        pltpu.sync_copy(work.at[pl.ds(r0, CHUNK)], bg0)
        _copy_half(bg0, sb, CHUNK, DH, 0)
        pltpu.sync_copy(sb, s_out.at[cid, pl.ds(r0, CHUNK)])
    if rem:
        r0 = off + nfull * CHUNK
        pltpu.sync_copy(work.at[pl.ds(r0, rem)], bg0.at[pl.ds(0, rem)])
        _copy_half(bg0, sb, rem, DH, 0)
        pltpu.sync_copy(sb.at[pl.ds(0, rem)], s_out.at[cid, pl.ds(r0, rem)])


def _make_sc_gather_sum():
    mesh = plsc.VectorSubcoreMesh(core_axis_name="c", subcore_axis_name="s")
    return pl.kernel(
        _sc_gather_sum_body,
        out_type=jax.ShapeDtypeStruct((NC, NACC, DH), jnp.float32),
        mesh=mesh,
        scratch_types=[
            pltpu.VMEM_SHARED((NACC, D), jnp.float32),
            pltpu.VMEM((IDXB, CHUNK), jnp.int32),
            pltpu.VMEM((IDXB, CHUNK), jnp.int32),
            pltpu.VMEM((CHUNK, D), jnp.float32),
            pltpu.VMEM((CHUNK, D), jnp.float32),
            pltpu.VMEM((CHUNK, D), jnp.float32),
            pltpu.VMEM((CHUNK, D), jnp.float32),
            pltpu.VMEM((CHUNK, DH), jnp.float32),
            pltpu.SemaphoreType.DMA,
            pltpu.SemaphoreType.DMA,
            pltpu.SemaphoreType.DMA,
            pltpu.SemaphoreType.DMA,
        ],
    )


def _sc_count_body(dst_hbm, c_out, acc, dst_v, ones_v, zeros_v):
    cid = lax.axis_index("c")
    sid = lax.axis_index("s")
    wid = sid * NC + cid

    _fill_zeros(zeros_v, CCHUNK, D)
    of = jnp.ones((16,), jnp.float32)

    @pl.loop(0, CCHUNK)
    def _(r):
        @pl.loop(0, D, step=16)
        def _(c0):
            ones_v[r, pl.ds(c0, 16)] = of

    _zero_acc(acc, zeros_v, sid)
    plsc.subcore_barrier()

    @pl.loop(0, CCH_PER_W // IDXB)
    def _(t):
        row0 = wid * CCH_PER_W + t * IDXB
        pltpu.sync_copy(dst_hbm.at[pl.ds(row0, IDXB)], dst_v)
        for j in range(IDXB):
            pltpu.sync_copy(ones_v, acc.at[dst_v.at[j]], add=True)

    plsc.subcore_barrier()

    off = sid * ROWS_PER_SUB
    pltpu.sync_copy(acc.at[pl.ds(off, ROWS_PER_SUB)],
                    c_out.at[cid, pl.ds(off, ROWS_PER_SUB)])


def _make_sc_count():
    mesh = plsc.VectorSubcoreMesh(core_axis_name="c", subcore_axis_name="s")
    return pl.kernel(
        _sc_count_body,
        out_type=jax.ShapeDtypeStruct((NC, NACC, D), jnp.float32),
        mesh=mesh,
        scratch_types=[
            pltpu.VMEM_SHARED((NACC, D), jnp.float32),
            pltpu.VMEM((IDXB, CCHUNK), jnp.int32),
            pltpu.VMEM((CCHUNK, D), jnp.float32),
            pltpu.VMEM((CCHUNK, D), jnp.float32),
        ],
    )


def _layer_math(s_ref, c_ref, h_ref, wl_ref, wr_ref, b_ref):
    s = jnp.concatenate([s_ref[0], s_ref[1]], axis=1)
    cnt = c_ref[0, :, 0:1] + c_ref[1, :, 0:1]
    agg = s / jnp.maximum(cnt, 1.0)
    t = (jnp.dot(agg, wl_ref[...], preferred_element_type=jnp.float32,
                 precision=lax.Precision.HIGHEST)
         + jnp.dot(h_ref[...], wr_ref[...], preferred_element_type=jnp.float32,
                   precision=lax.Precision.HIGHEST)
         + b_ref[...])
    return jnp.where(t >= 0, t, 0.01 * t)


def _combine_body(s_ref, c_ref, h_ref, wl_ref, wr_ref, b_ref, o_ref):
    o_ref[...] = _layer_math(s_ref, c_ref, h_ref, wl_ref, wr_ref, b_ref)


BLK = 632
NBLK = NACC // BLK


def _combine(s, c_part, h, wl, wr, b):
    return pl.pallas_call(
        _combine_body,
        grid=(NBLK,),
        in_specs=[
            pl.BlockSpec((NC, BLK, DH), lambda i: (0, i, 0)),
            pl.BlockSpec((NC, BLK, 16), lambda i: (0, i, 0)),
            pl.BlockSpec((BLK, D), lambda i: (i, 0)),
            pl.BlockSpec((D, D), lambda i: (0, 0)),
            pl.BlockSpec((D, D), lambda i: (0, 0)),
            pl.BlockSpec((1, D), lambda i: (0, 0)),
        ],
        out_specs=pl.BlockSpec((BLK, D), lambda i: (i, 0)),
        out_shape=jax.ShapeDtypeStruct((NACC, D), jnp.float32),
    )(s, c_part, h, wl, wr, b)


FBLK = 1000
FNBLK = N // FBLK


def _combine_final_body(s_ref, c_ref, h_ref, wl_ref, wr_ref, b_ref,
                        wfc_ref, bfc_ref, o_ref, mx_ref):
    i = pl.program_id(0)

    @pl.when(i == 0)
    def _():
        mx_ref[...] = jnp.full((8, D), -jnp.inf, jnp.float32)

    hb = _layer_math(s_ref, c_ref, h_ref, wl_ref, wr_ref, b_ref)
    mx_ref[...] = jnp.maximum(mx_ref[...],
                              jnp.max(hb.reshape(FBLK // 8, 8, D), axis=0))

    @pl.when(i == FNBLK - 1)
    def _():
        pooled = jnp.max(mx_ref[...], axis=0).reshape(1, D)
        o_ref[...] = (jnp.dot(pooled, wfc_ref[...],
                              preferred_element_type=jnp.float32,
                              precision=lax.Precision.HIGHEST)
                      + bfc_ref[...])


def _combine_final(s, c_part, h, wl, wr, b, wfc, bfc):
    return pl.pallas_call(
        _combine_final_body,
        grid=(FNBLK,),
        in_specs=[
            pl.BlockSpec((NC, FBLK, DH), lambda i: (0, i, 0)),
            pl.BlockSpec((NC, FBLK, 16), lambda i: (0, i, 0)),
            pl.BlockSpec((FBLK, D), lambda i: (i, 0)),
            pl.BlockSpec((D, D), lambda i: (0, 0)),
            pl.BlockSpec((D, D), lambda i: (0, 0)),
            pl.BlockSpec((1, D), lambda i: (0, 0)),
            pl.BlockSpec((D, D_OUT), lambda i: (0, 0)),
            pl.BlockSpec((1, D_OUT), lambda i: (0, 0)),
        ],
        out_specs=pl.BlockSpec((1, D_OUT), lambda i: (0, 0)),
        out_shape=jax.ShapeDtypeStruct((1, D_OUT), jnp.float32),
        scratch_shapes=[pltpu.VMEM((8, D), jnp.float32)],
    )(s, c_part, h, wl, wr, b, wfc, bfc)


def kernel(x, edge_index, batch, W1l, W1r, b1, W2l, W2r, b2, W3l, W3r, b3,
           W_fc, b_fc):
    src = edge_index[0]
    dst = edge_index[1]
    pad = EPAD - E
    src2d = jnp.concatenate(
        [src, jnp.zeros((pad,), jnp.int32)]).reshape(NCHUNKS, CHUNK)
    dst2d = jnp.concatenate(
        [dst, jnp.full((pad,), N, jnp.int32)]).reshape(NCHUNKS, CHUNK)
    dst2d_w = dst2d.reshape(CNCHUNKS, CCHUNK)

    xp = jnp.zeros((NACC, D), jnp.float32).at[:N].set(x)

    gather_sum = _make_sc_gather_sum()
    count = _make_sc_count()

    c_full = count(dst2d_w)
    c_part = c_full[:, :, :16]

    s1 = gather_sum(xp, src2d, dst2d)
    h1 = _combine(s1, c_part, xp, W1l, W1r, b1.reshape(1, D))
    s2 = gather_sum(h1, src2d, dst2d)
    h2 = _combine(s2, c_part, h1, W2l, W2r, b2.reshape(1, D))
    s3 = gather_sum(h2, src2d, dst2d)
    return _combine_final(s3, c_part, h2, W3l, W3r, b3.reshape(1, D),
                          W_fc, b_fc.reshape(1, D_OUT))


# speedup vs baseline: 1.5033x; 1.0648x over previous
import jax
import jax.numpy as jnp
from jax import lax
from jax.experimental import pallas as pl
from jax.experimental.pallas import tpu as pltpu
from jax.experimental.pallas import tpu_sc as plsc

N = 10000
E = 320000
D = 128
DH = 64
D_OUT = 16

NC = 2
NS = 16
NW = NC * NS
CHUNK = 64
NCHUNKS = 5120
EPAD = NCHUNKS * CHUNK
CH_PER_T = NCHUNKS // NS
IDXB = 16
NACC = 10112
ROWS_PER_SUB = NACC // NS

CCHUNK = 128
CNCHUNKS = EPAD // CCHUNK
CCH_PER_W = CNCHUNKS // NW


def _fill_zeros(ref, rows, width):
    zf = jnp.zeros((16,), jnp.float32)

    @pl.loop(0, rows)
    def _(r):
        @pl.loop(0, width, step=16)
        def _(c0):
            ref[r, pl.ds(c0, 16)] = zf


def _zero_acc(acc, zeros_v, sid):
    blk = zeros_v.shape[0]

    @pl.loop(0, ROWS_PER_SUB // blk)
    def _(t):
        pltpu.sync_copy(zeros_v, acc.at[pl.ds(sid * ROWS_PER_SUB + t * blk, blk)])
    rem = ROWS_PER_SUB % blk
    if rem:
        off = sid * ROWS_PER_SUB + (ROWS_PER_SUB // blk) * blk
        pltpu.sync_copy(zeros_v.at[pl.ds(0, rem)], acc.at[pl.ds(off, rem)])


def _copy_half(srcb, dstb, rows, src_lane0, dst_lane0):
    @pl.loop(0, rows)
    def _(r):
        for k in range(DH // 16):
            dstb[r, pl.ds(dst_lane0 + k * 16, 16)] = (
                srcb[r, pl.ds(src_lane0 + k * 16, 16)])


def _sc_gather_sum_body(h_hbm, src_hbm, dst_hbm, s_out, work,
                        src_v, dst_v, bg0, bg1, ba0, ba1, sb,
                        g0, g1, a0, a1):
    cid = lax.axis_index("c")
    sid = lax.axis_index("s")

    bg = (bg0, bg1)
    ba = (ba0, ba1)
    gsem = (g0, g1)
    asem = (a0, a1)

    off = sid * ROWS_PER_SUB
    nfull = ROWS_PER_SUB // CHUNK
    rem = ROWS_PER_SUB % CHUNK

    _fill_zeros(ba0, CHUNK, D)

    @pl.loop(0, nfull)
    def _(bk):
        r0 = off + bk * CHUNK
        pltpu.sync_copy(h_hbm.at[pl.ds(r0, CHUNK)], bg0)

        @pl.when(cid == 0)
        def _():
            _copy_half(bg0, ba0, CHUNK, 0, 0)

        @pl.when(cid == 1)
        def _():
            _copy_half(bg0, ba0, CHUNK, DH, 0)
        pltpu.sync_copy(ba0, work.at[pl.ds(r0, CHUNK)])
    if rem:
        r0 = off + nfull * CHUNK
        pltpu.sync_copy(h_hbm.at[pl.ds(r0, rem)], bg0.at[pl.ds(0, rem)])

        @pl.when(cid == 0)
        def _():
            _copy_half(bg0, ba0, rem, 0, 0)

        @pl.when(cid == 1)
        def _():
            _copy_half(bg0, ba0, rem, DH, 0)
        pltpu.sync_copy(ba0.at[pl.ds(0, rem)], work.at[pl.ds(r0, rem)])

    _fill_zeros(ba0, CHUNK, D)
    _fill_zeros(ba1, CHUNK, D)
    plsc.subcore_barrier()

    @pl.loop(0, CH_PER_T // IDXB)
    def _(t):
        row0 = sid * CH_PER_T + t * IDXB
        pltpu.sync_copy(src_hbm.at[pl.ds(row0, IDXB)], src_v)
        pltpu.sync_copy(dst_hbm.at[pl.ds(row0, IDXB)], dst_v)
        pend_g = [None, None]
        pend_a = [None, None]
        pend_g[0] = pltpu.async_copy(work.at[src_v.at[0]], bg0, g0)
        for j in range(IDXB):
            b = j % 2
            if j + 1 < IDXB:
                nb = (j + 1) % 2
                pend_g[nb] = pltpu.async_copy(work.at[src_v.at[j + 1]],
                                              bg[nb], gsem[nb])
            pend_g[b].wait()
            if pend_a[b] is not None:
                pend_a[b].wait()
            _copy_half(bg[b], ba[b], CHUNK, 0, DH)
            pend_a[b] = pltpu.async_copy(ba[b], work.at[dst_v.at[j]],
                                         asem[b], add=True)
        pend_a[0].wait()
        pend_a[1].wait()

    plsc.subcore_barrier()

    @pl.loop(0, nfull)
    def _(bk):
        r0 = off + bk * CHUNK
        pltpu.sync_copy(work.at[pl.ds(r0, CHUNK)], bg0)
        _copy_half(bg0, sb, CHUNK, DH, 0)
        pltpu.sync_copy(sb, s_out.at[cid, pl.ds(r0, CHUNK)])
    if rem:
        r0 = off + nfull * CHUNK
        pltpu.sync_copy(work.at[pl.ds(r0, rem)], bg0.at[pl.ds(0, rem)])
        _copy_half(bg0, sb, rem, DH, 0)
        pltpu.sync_copy(sb.at[pl.ds(0, rem)], s_out.at[cid, pl.ds(r0, rem)])


def _make_sc_gather_sum():
    mesh = plsc.VectorSubcoreMesh(core_axis_name="c", subcore_axis_name="s")
    return pl.kernel(
        _sc_gather_sum_body,
        out_type=jax.ShapeDtypeStruct((NC, NACC, DH), jnp.float32),
        mesh=mesh,
        scratch_types=[
            pltpu.VMEM_SHARED((NACC, D), jnp.float32),
            pltpu.VMEM((IDXB, CHUNK), jnp.int32),
            pltpu.VMEM((IDXB, CHUNK), jnp.int32),
            pltpu.VMEM((CHUNK, D), jnp.float32),
            pltpu.VMEM((CHUNK, D), jnp.float32),
            pltpu.VMEM((CHUNK, D), jnp.float32),
            pltpu.VMEM((CHUNK, D), jnp.float32),
            pltpu.VMEM((CHUNK, DH), jnp.float32),
            pltpu.SemaphoreType.DMA,
            pltpu.SemaphoreType.DMA,
            pltpu.SemaphoreType.DMA,
            pltpu.SemaphoreType.DMA,
        ],
    )


def _sc_count_body(dst_hbm, c_out, acc, dst_v, ones_v, zeros_v):
    cid = lax.axis_index("c")
    sid = lax.axis_index("s")
    wid = sid * NC + cid

    _fill_zeros(zeros_v, CCHUNK, D)
    of = jnp.ones((16,), jnp.float32)

    @pl.loop(0, CCHUNK)
    def _(r):
        @pl.loop(0, D, step=16)
        def _(c0):
            ones_v[r, pl.ds(c0, 16)] = of

    _zero_acc(acc, zeros_v, sid)
    plsc.subcore_barrier()

    @pl.loop(0, CCH_PER_W // IDXB)
    def _(t):
        row0 = wid * CCH_PER_W + t * IDXB
        pltpu.sync_copy(dst_hbm.at[pl.ds(row0, IDXB)], dst_v)
        for j in range(IDXB):
            pltpu.sync_copy(ones_v, acc.at[dst_v.at[j]], add=True)

    plsc.subcore_barrier()

    off = sid * ROWS_PER_SUB
    pltpu.sync_copy(acc.at[pl.ds(off, ROWS_PER_SUB)],
                    c_out.at[cid, pl.ds(off, ROWS_PER_SUB)])


def _make_sc_count():
    mesh = plsc.VectorSubcoreMesh(core_axis_name="c", subcore_axis_name="s")
    return pl.kernel(
        _sc_count_body,
        out_type=jax.ShapeDtypeStruct((NC, NACC, D), jnp.float32),
        mesh=mesh,
        scratch_types=[
            pltpu.VMEM_SHARED((NACC, D), jnp.float32),
            pltpu.VMEM((IDXB, CCHUNK), jnp.int32),
            pltpu.VMEM((CCHUNK, D), jnp.float32),
            pltpu.VMEM((CCHUNK, D), jnp.float32),
        ],
    )


def _layer_math(s_ref, c_ref, h_ref, wl_ref, wr_ref, b_ref):
    s = jnp.concatenate([s_ref[0], s_ref[1]], axis=1)
    cnt = c_ref[0, :, 0:1] + c_ref[1, :, 0:1]
    agg = s / jnp.maximum(cnt, 1.0)
    t = (jnp.dot(agg, wl_ref[...], preferred_element_type=jnp.float32,
                 precision=lax.Precision.HIGHEST)
         + jnp.dot(h_ref[...], wr_ref[...], preferred_element_type=jnp.float32,
                   precision=lax.Precision.HIGHEST)
         + b_ref[...])
    return jnp.where(t >= 0, t, 0.01 * t)


def _combine_body(s_ref, c_ref, h_ref, wl_ref, wr_ref, b_ref, o_ref):
    o_ref[...] = _layer_math(s_ref, c_ref, h_ref, wl_ref, wr_ref, b_ref)


BLK = 632
NBLK = NACC // BLK


def _combine(s, c_part, h, wl, wr, b):
    return pl.pallas_call(
        _combine_body,
        grid=(NBLK,),
        in_specs=[
            pl.BlockSpec((NC, BLK, DH), lambda i: (0, i, 0)),
            pl.BlockSpec((NC, BLK, 16), lambda i: (0, i, 0)),
            pl.BlockSpec((BLK, D), lambda i: (i, 0)),
            pl.BlockSpec((D, D), lambda i: (0, 0)),
            pl.BlockSpec((D, D), lambda i: (0, 0)),
            pl.BlockSpec((1, D), lambda i: (0, 0)),
        ],
        out_specs=pl.BlockSpec((BLK, D), lambda i: (i, 0)),
        out_shape=jax.ShapeDtypeStruct((NACC, D), jnp.float32),
    )(s, c_part, h, wl, wr, b)


FBLK = 1000
FNBLK = N // FBLK


def _combine_final_body(s_ref, c_ref, h_ref, wl_ref, wr_ref, b_ref,
                        wfc_ref, bfc_ref, o_ref, mx_ref):
    i = pl.program_id(0)

    @pl.when(i == 0)
    def _():
        mx_ref[...] = jnp.full((8, D), -jnp.inf, jnp.float32)

    hb = _layer_math(s_ref, c_ref, h_ref, wl_ref, wr_ref, b_ref)
    mx_ref[...] = jnp.maximum(mx_ref[...],
                              jnp.max(hb.reshape(FBLK // 8, 8, D), axis=0))

    @pl.when(i == FNBLK - 1)
    def _():
        pooled = jnp.max(mx_ref[...], axis=0).reshape(1, D)
        o_ref[...] = (jnp.dot(pooled, wfc_ref[...],
                              preferred_element_type=jnp.float32,
                              precision=lax.Precision.HIGHEST)
                      + bfc_ref[...])


def _combine_final(s, c_part, h, wl, wr, b, wfc, bfc):
    return pl.pallas_call(
        _combine_final_body,
        grid=(FNBLK,),
        in_specs=[
            pl.BlockSpec((NC, FBLK, DH), lambda i: (0, i, 0)),
            pl.BlockSpec((NC, FBLK, 16), lambda i: (0, i, 0)),
            pl.BlockSpec((FBLK, D), lambda i: (i, 0)),
            pl.BlockSpec((D, D), lambda i: (0, 0)),
            pl.BlockSpec((D, D), lambda i: (0, 0)),
            pl.BlockSpec((1, D), lambda i: (0, 0)),
            pl.BlockSpec((D, D_OUT), lambda i: (0, 0)),
            pl.BlockSpec((1, D_OUT), lambda i: (0, 0)),
        ],
        out_specs=pl.BlockSpec((1, D_OUT), lambda i: (0, 0)),
        out_shape=jax.ShapeDtypeStruct((1, D_OUT), jnp.float32),
        scratch_shapes=[pltpu.VMEM((8, D), jnp.float32)],
    )(s, c_part, h, wl, wr, b, wfc, bfc)


def kernel(x, edge_index, batch, W1l, W1r, b1, W2l, W2r, b2, W3l, W3r, b3,
           W_fc, b_fc):
    src = edge_index[0]
    dst = edge_index[1]
    pad = EPAD - E
    src2d = jnp.concatenate(
        [src, jnp.zeros((pad,), jnp.int32)]).reshape(NCHUNKS, CHUNK)
    dst2d = jnp.concatenate(
        [dst, jnp.full((pad,), N, jnp.int32)]).reshape(NCHUNKS, CHUNK)
    dst2d_w = dst2d.reshape(CNCHUNKS, CCHUNK)

    xp = jnp.zeros((NACC, D), jnp.float32).at[:N].set(x)

    gather_sum = _make_sc_gather_sum()
    count = _make_sc_count()

    c_full = count(dst2d_w)
    c_part = c_full[:, :, :16]

    s1 = gather_sum(xp, src2d, dst2d)
    h1 = _combine(s1, c_part, xp, W1l, W1r, b1.reshape(1, D))
    s2 = gather_sum(h1, src2d, dst2d)
    h2 = _combine(s2, c_part, h1, W2l, W2r, b2.reshape(1, D))
    s3 = gather_sum(h2, src2d, dst2d)
    return _combine_final(s3, c_part, h2, W3l, W3r, b3.reshape(1, D),
                          W_fc, b_fc.reshape(1, D_OUT))


# speedup vs baseline: 1.5643x; 1.0406x over previous
import jax
import jax.numpy as jnp
from jax import lax
from jax.experimental import pallas as pl
from jax.experimental.pallas import tpu as pltpu
from jax.experimental.pallas import tpu_sc as plsc

N = 10000
E = 320000
D = 128
DH = 64
D_OUT = 16

NC = 2
NS = 16
NW = NC * NS
CHUNK = 64
NCHUNKS = 5120
EPAD = NCHUNKS * CHUNK
CH_PER_T = NCHUNKS // NS
IDXB = 16
NACC = 10112
ROWS_PER_SUB = NACC // NS

CCHUNK = 128
CNCHUNKS = EPAD // CCHUNK
CCH_PER_W = CNCHUNKS // NW


def _fill_zeros(ref, rows, width):
    zf = jnp.zeros((16,), jnp.float32)

    @pl.loop(0, rows)
    def _(r):
        @pl.loop(0, width, step=16)
        def _(c0):
            ref[r, pl.ds(c0, 16)] = zf


def _zero_acc(acc, zeros_v, sid):
    blk = zeros_v.shape[0]

    @pl.loop(0, ROWS_PER_SUB // blk)
    def _(t):
        pltpu.sync_copy(zeros_v, acc.at[pl.ds(sid * ROWS_PER_SUB + t * blk, blk)])
    rem = ROWS_PER_SUB % blk
    if rem:
        off = sid * ROWS_PER_SUB + (ROWS_PER_SUB // blk) * blk
        pltpu.sync_copy(zeros_v.at[pl.ds(0, rem)], acc.at[pl.ds(off, rem)])


def _copy_half(srcb, dstb, rows, src_lane0, dst_lane0):
    @pl.loop(0, rows)
    def _(r):
        for k in range(DH // 16):
            dstb[r, pl.ds(dst_lane0 + k * 16, 16)] = (
                srcb[r, pl.ds(src_lane0 + k * 16, 16)])


def _sc_gather_sum_body(h_hbm, sd_hbm, s_out, work,
                        sd_v, bg0, bg1, ba0, ba1, sb,
                        g0, g1, a0, a1):
    cid = lax.axis_index("c")
    sid = lax.axis_index("s")

    bg = (bg0, bg1)
    ba = (ba0, ba1)
    gsem = (g0, g1)
    asem = (a0, a1)

    off = sid * ROWS_PER_SUB
    nfull = ROWS_PER_SUB // CHUNK
    rem = ROWS_PER_SUB % CHUNK

    _fill_zeros(ba0, CHUNK, D)

    @pl.loop(0, nfull)
    def _(bk):
        r0 = off + bk * CHUNK
        pltpu.sync_copy(h_hbm.at[pl.ds(r0, CHUNK)], bg0)

        @pl.when(cid == 0)
        def _():
            _copy_half(bg0, ba0, CHUNK, 0, 0)

        @pl.when(cid == 1)
        def _():
            _copy_half(bg0, ba0, CHUNK, DH, 0)
        pltpu.sync_copy(ba0, work.at[pl.ds(r0, CHUNK)])
    if rem:
        r0 = off + nfull * CHUNK
        pltpu.sync_copy(h_hbm.at[pl.ds(r0, rem)], bg0.at[pl.ds(0, rem)])

        @pl.when(cid == 0)
        def _():
            _copy_half(bg0, ba0, rem, 0, 0)

        @pl.when(cid == 1)
        def _():
            _copy_half(bg0, ba0, rem, DH, 0)
        pltpu.sync_copy(ba0.at[pl.ds(0, rem)], work.at[pl.ds(r0, rem)])

    _fill_zeros(ba0, CHUNK, D)
    _fill_zeros(ba1, CHUNK, D)
    plsc.subcore_barrier()

    @pl.loop(0, CH_PER_T // IDXB)
    def _(t):
        row0 = sid * CH_PER_T + t * IDXB
        pltpu.sync_copy(sd_hbm.at[pl.ds(row0, IDXB)], sd_v)
        pend_g = [None, None]
        pend_a = [None, None]
        pend_g[0] = pltpu.async_copy(work.at[sd_v.at[0, 0]], bg0, g0)
        for j in range(IDXB):
            b = j % 2
            if j + 1 < IDXB:
                nb = (j + 1) % 2
                pend_g[nb] = pltpu.async_copy(work.at[sd_v.at[j + 1, 0]],
                                              bg[nb], gsem[nb])
            pend_g[b].wait()
            if pend_a[b] is not None:
                pend_a[b].wait()
            _copy_half(bg[b], ba[b], CHUNK, 0, DH)
            pend_a[b] = pltpu.async_copy(ba[b], work.at[sd_v.at[j, 1]],
                                         asem[b], add=True)
        pend_a[0].wait()
        pend_a[1].wait()

    plsc.subcore_barrier()

    @pl.loop(0, nfull)
    def _(bk):
        r0 = off + bk * CHUNK
        pltpu.sync_copy(work.at[pl.ds(r0, CHUNK)], bg0)
        _copy_half(bg0, sb, CHUNK, DH, 0)
        pltpu.sync_copy(sb, s_out.at[cid, pl.ds(r0, CHUNK)])
    if rem:
        r0 = off + nfull * CHUNK
        pltpu.sync_copy(work.at[pl.ds(r0, rem)], bg0.at[pl.ds(0, rem)])
        _copy_half(bg0, sb, rem, DH, 0)
        pltpu.sync_copy(sb.at[pl.ds(0, rem)], s_out.at[cid, pl.ds(r0, rem)])


def _make_sc_gather_sum():
    mesh = plsc.VectorSubcoreMesh(core_axis_name="c", subcore_axis_name="s")
    return pl.kernel(
        _sc_gather_sum_body,
        out_type=jax.ShapeDtypeStruct((NC, NACC, DH), jnp.float32),
        mesh=mesh,
        scratch_types=[
            pltpu.VMEM_SHARED((NACC, D), jnp.float32),
            pltpu.VMEM((IDXB, 2, CHUNK), jnp.int32),
            pltpu.VMEM((CHUNK, D), jnp.float32),
            pltpu.VMEM((CHUNK, D), jnp.float32),
            pltpu.VMEM((CHUNK, D), jnp.float32),
            pltpu.VMEM((CHUNK, D), jnp.float32),
            pltpu.VMEM((CHUNK, DH), jnp.float32),
            pltpu.SemaphoreType.DMA,
            pltpu.SemaphoreType.DMA,
            pltpu.SemaphoreType.DMA,
            pltpu.SemaphoreType.DMA,
        ],
    )


def _sc_count_body(dst_hbm, c_out, acc, dst_v, ones_v, zeros_v):
    cid = lax.axis_index("c")
    sid = lax.axis_index("s")
    wid = sid * NC + cid

    _fill_zeros(zeros_v, CCHUNK, D)
    of = jnp.ones((16,), jnp.float32)

    @pl.loop(0, CCHUNK)
    def _(r):
        @pl.loop(0, D, step=16)
        def _(c0):
            ones_v[r, pl.ds(c0, 16)] = of

    _zero_acc(acc, zeros_v, sid)
    plsc.subcore_barrier()

    @pl.loop(0, CCH_PER_W // IDXB)
    def _(t):
        row0 = wid * CCH_PER_W + t * IDXB
        pltpu.sync_copy(dst_hbm.at[pl.ds(row0, IDXB)], dst_v)
        for j in range(IDXB):
            pltpu.sync_copy(ones_v, acc.at[dst_v.at[j]], add=True)

    plsc.subcore_barrier()

    off = sid * ROWS_PER_SUB
    pltpu.sync_copy(acc.at[pl.ds(off, ROWS_PER_SUB)],
                    c_out.at[cid, pl.ds(off, ROWS_PER_SUB)])


def _make_sc_count():
    mesh = plsc.VectorSubcoreMesh(core_axis_name="c", subcore_axis_name="s")
    return pl.kernel(
        _sc_count_body,
        out_type=jax.ShapeDtypeStruct((NC, NACC, D), jnp.float32),
        mesh=mesh,
        scratch_types=[
            pltpu.VMEM_SHARED((NACC, D), jnp.float32),
            pltpu.VMEM((IDXB, CCHUNK), jnp.int32),
            pltpu.VMEM((CCHUNK, D), jnp.float32),
            pltpu.VMEM((CCHUNK, D), jnp.float32),
        ],
    )


def _layer_math(s_ref, c_ref, h_ref, wl_ref, wr_ref, b_ref):
    s = jnp.concatenate([s_ref[0], s_ref[1]], axis=1)
    cnt = c_ref[0, :, 0:1] + c_ref[1, :, 0:1]
    agg = s / jnp.maximum(cnt, 1.0)
    t = (jnp.dot(agg, wl_ref[...], preferred_element_type=jnp.float32,
                 precision=lax.Precision.HIGHEST)
         + jnp.dot(h_ref[...], wr_ref[...], preferred_element_type=jnp.float32,
                   precision=lax.Precision.HIGHEST)
         + b_ref[...])
    return jnp.where(t >= 0, t, 0.01 * t)


def _combine_body(s_ref, c_ref, h_ref, wl_ref, wr_ref, b_ref, o_ref):
    o_ref[...] = _layer_math(s_ref, c_ref, h_ref, wl_ref, wr_ref, b_ref)


BLK = 632
NBLK = NACC // BLK


def _combine(s, c_part, h, wl, wr, b):
    return pl.pallas_call(
        _combine_body,
        grid=(NBLK,),
        in_specs=[
            pl.BlockSpec((NC, BLK, DH), lambda i: (0, i, 0)),
            pl.BlockSpec((NC, BLK, 16), lambda i: (0, i, 0)),
            pl.BlockSpec((BLK, D), lambda i: (i, 0)),
            pl.BlockSpec((D, D), lambda i: (0, 0)),
            pl.BlockSpec((D, D), lambda i: (0, 0)),
            pl.BlockSpec((1, D), lambda i: (0, 0)),
        ],
        out_specs=pl.BlockSpec((BLK, D), lambda i: (i, 0)),
        out_shape=jax.ShapeDtypeStruct((NACC, D), jnp.float32),
    )(s, c_part, h, wl, wr, b)


FBLK = 1000
FNBLK = N // FBLK


def _combine_final_body(s_ref, c_ref, h_ref, wl_ref, wr_ref, b_ref,
                        wfc_ref, bfc_ref, o_ref, mx_ref):
    i = pl.program_id(0)

    @pl.when(i == 0)
    def _():
        mx_ref[...] = jnp.full((8, D), -jnp.inf, jnp.float32)

    hb = _layer_math(s_ref, c_ref, h_ref, wl_ref, wr_ref, b_ref)
    mx_ref[...] = jnp.maximum(mx_ref[...],
                              jnp.max(hb.reshape(FBLK // 8, 8, D), axis=0))

    @pl.when(i == FNBLK - 1)
    def _():
        pooled = jnp.max(mx_ref[...], axis=0).reshape(1, D)
        o_ref[...] = (jnp.dot(pooled, wfc_ref[...],
                              preferred_element_type=jnp.float32,
                              precision=lax.Precision.HIGHEST)
                      + bfc_ref[...])


def _combine_final(s, c_part, h, wl, wr, b, wfc, bfc):
    return pl.pallas_call(
        _combine_final_body,
        grid=(FNBLK,),
        in_specs=[
            pl.BlockSpec((NC, FBLK, DH), lambda i: (0, i, 0)),
            pl.BlockSpec((NC, FBLK, 16), lambda i: (0, i, 0)),
            pl.BlockSpec((FBLK, D), lambda i: (i, 0)),
            pl.BlockSpec((D, D), lambda i: (0, 0)),
            pl.BlockSpec((D, D), lambda i: (0, 0)),
            pl.BlockSpec((1, D), lambda i: (0, 0)),
            pl.BlockSpec((D, D_OUT), lambda i: (0, 0)),
            pl.BlockSpec((1, D_OUT), lambda i: (0, 0)),
        ],
        out_specs=pl.BlockSpec((1, D_OUT), lambda i: (0, 0)),
        out_shape=jax.ShapeDtypeStruct((1, D_OUT), jnp.float32),
        scratch_shapes=[pltpu.VMEM((8, D), jnp.float32)],
    )(s, c_part, h, wl, wr, b, wfc, bfc)


def kernel(x, edge_index, batch, W1l, W1r, b1, W2l, W2r, b2, W3l, W3r, b3,
           W_fc, b_fc):
    src = edge_index[0]
    dst = edge_index[1]
    pad = EPAD - E
    src2d = jnp.concatenate(
        [src, jnp.zeros((pad,), jnp.int32)]).reshape(NCHUNKS, CHUNK)
    dst2d = jnp.concatenate(
        [dst, jnp.full((pad,), N, jnp.int32)]).reshape(NCHUNKS, CHUNK)
    dst2d_w = dst2d.reshape(CNCHUNKS, CCHUNK)
    sd3d = jnp.stack([src2d, dst2d], axis=1)

    xp = jnp.zeros((NACC, D), jnp.float32).at[:N].set(x)

    gather_sum = _make_sc_gather_sum()
    count = _make_sc_count()

    c_full = count(dst2d_w)
    c_part = c_full[:, :, :16]

    s1 = gather_sum(xp, sd3d)
    h1 = _combine(s1, c_part, xp, W1l, W1r, b1.reshape(1, D))
    s2 = gather_sum(h1, sd3d)
    h2 = _combine(s2, c_part, h1, W2l, W2r, b2.reshape(1, D))
    s3 = gather_sum(h2, sd3d)
    return _combine_final(s3, c_part, h2, W3l, W3r, b3.reshape(1, D),
                          W_fc, b_fc.reshape(1, D_OUT))
